# baseline (speedup 1.0000x reference)
import functools

import jax
import jax.numpy as jnp
from jax import lax
from jax.experimental import pallas as pl
from jax.experimental.pallas import tpu as pltpu
from jax.experimental.pallas import tpu_sc as plsc

_ROWS_PER_BLOCK = 1024
_CHUNK = 80


def _argmax_body(x_ref, tok_ref):
    xb = x_ref[...]
    vocab = xb.shape[-1]
    m = jnp.max(xb, axis=-1, keepdims=True)
    col = jax.lax.broadcasted_iota(jnp.int32, xb.shape, 1)
    tok_ref[...] = jnp.min(jnp.where(xb == m, col, vocab), axis=-1)


def _compute_tokens(x2):
    rows, vocab = x2.shape
    grid = rows // _ROWS_PER_BLOCK
    return pl.pallas_call(
        _argmax_body,
        grid=(grid,),
        in_specs=[pl.BlockSpec((_ROWS_PER_BLOCK, vocab), lambda i: (i, 0))],
        out_specs=pl.BlockSpec((_ROWS_PER_BLOCK,), lambda i: (i,)),
        out_shape=jax.ShapeDtypeStruct((rows,), jnp.int32),
    )(x2)


def _make_gather(rows, emb, n_workers, n_chunks):
    bpw = rows // n_workers

    def _gather_body(tok_hbm, table_hbm, out_hbm, idx_v, rows_v, sem):
        wid = lax.axis_index("s") * 2 + lax.axis_index("c")
        pltpu.sync_copy(tok_hbm.at[pl.ds(wid * bpw, bpw)], idx_v)
        copies = [
            pltpu.async_copy(
                table_hbm.at[idx_v.at[pl.ds(j * _CHUNK, _CHUNK)]],
                rows_v.at[pl.ds(j * _CHUNK, _CHUNK)],
                sem,
            )
            for j in range(n_chunks)
        ]
        for cp in copies:
            cp.wait()
        pltpu.sync_copy(rows_v, out_hbm.at[pl.ds(wid * bpw, bpw)])

    mesh = plsc.VectorSubcoreMesh(core_axis_name="c", subcore_axis_name="s")
    return pl.kernel(
        _gather_body,
        mesh=mesh,
        compiler_params=pltpu.CompilerParams(use_tc_tiling_on_sc=False),
        out_type=jax.ShapeDtypeStruct((rows, emb), jnp.float32),
        scratch_types=[
            pltpu.VMEM((bpw,), jnp.int32),
            pltpu.VMEM((bpw, emb), jnp.float32),
            pltpu.SemaphoreType.DMA,
        ],
    )


def kernel(x, dictionary):
    b, n, vocab = x.shape
    emb = dictionary.shape[1]
    rows = b * n
    n_workers = 32
    n_chunks = rows // (n_workers * _CHUNK)

    tokens = _compute_tokens(x.reshape(rows, vocab))
    out = _make_gather(rows, emb, n_workers, n_chunks)(tokens, dictionary)
    return out.reshape(b, n, emb)


# speedup vs baseline: 1.2458x; 1.2458x over previous
import functools

import jax
import jax.numpy as jnp
from jax import lax
from jax.experimental import pallas as pl
from jax.experimental.pallas import tpu as pltpu
from jax.experimental.pallas import tpu_sc as plsc

_BATCH_PER_BLOCK = 16
_CHUNK = 80


def _argmax_body(x_ref, tok_ref):
    xb = x_ref[...]
    vocab = xb.shape[-1]
    m = jnp.max(xb, axis=-1, keepdims=True)
    col = jax.lax.broadcasted_iota(jnp.int32, xb.shape, 2)
    tok_ref[...] = jnp.min(jnp.where(xb == m, col, vocab), axis=-1)


def _compute_tokens(x):
    b, n, vocab = x.shape
    grid = b // _BATCH_PER_BLOCK
    return pl.pallas_call(
        _argmax_body,
        grid=(grid,),
        in_specs=[pl.BlockSpec((_BATCH_PER_BLOCK, n, vocab), lambda i: (i, 0, 0))],
        out_specs=pl.BlockSpec((_BATCH_PER_BLOCK, n), lambda i: (i, 0)),
        out_shape=jax.ShapeDtypeStruct((b, n), jnp.int32),
    )(x)


def _make_gather(rows, emb, n_workers, n_chunks):
    bpw = rows // n_workers

    def _gather_body(tok_hbm, table_hbm, out_hbm, idx_v, rows_v, sem):
        wid = lax.axis_index("s") * 2 + lax.axis_index("c")
        pltpu.sync_copy(tok_hbm.at[pl.ds(wid * bpw, bpw)], idx_v)
        copies = [
            pltpu.async_copy(
                table_hbm.at[idx_v.at[pl.ds(j * _CHUNK, _CHUNK)]],
                rows_v.at[pl.ds(j * _CHUNK, _CHUNK)],
                sem,
            )
            for j in range(n_chunks)
        ]
        for cp in copies:
            cp.wait()
        pltpu.sync_copy(rows_v, out_hbm.at[pl.ds(wid * bpw, bpw)])

    mesh = plsc.VectorSubcoreMesh(core_axis_name="c", subcore_axis_name="s")
    return pl.kernel(
        _gather_body,
        mesh=mesh,
        compiler_params=pltpu.CompilerParams(use_tc_tiling_on_sc=False),
        out_type=jax.ShapeDtypeStruct((rows, emb), jnp.float32),
        scratch_types=[
            pltpu.VMEM((bpw,), jnp.int32),
            pltpu.VMEM((bpw, emb), jnp.float32),
            pltpu.SemaphoreType.DMA,
        ],
    )


def kernel(x, dictionary):
    b, n, vocab = x.shape
    emb = dictionary.shape[1]
    rows = b * n
    n_workers = 32
    n_chunks = rows // (n_workers * _CHUNK)

    tokens = _compute_tokens(x).reshape(rows)
    out = _make_gather(rows, emb, n_workers, n_chunks)(tokens, dictionary)
    return out.reshape(b, n, emb)


# speedup vs baseline: 1.3312x; 1.0686x over previous
import functools

import jax
import jax.numpy as jnp
from jax import lax
from jax.experimental import pallas as pl
from jax.experimental.pallas import tpu as pltpu
from jax.experimental.pallas import tpu_sc as plsc

_BATCH_PER_BLOCK = 64
_CHUNK = 80


def _argmax_body(x_ref, tok_ref):
    xb = x_ref[...]
    vocab = xb.shape[-1]
    m = jnp.max(xb, axis=-1, keepdims=True)
    col = jax.lax.broadcasted_iota(jnp.int32, xb.shape, 2)
    tok_ref[...] = jnp.min(jnp.where(xb == m, col, vocab), axis=-1)


def _compute_tokens(x):
    b, n, vocab = x.shape
    grid = b // _BATCH_PER_BLOCK
    return pl.pallas_call(
        _argmax_body,
        grid=(grid,),
        in_specs=[pl.BlockSpec((_BATCH_PER_BLOCK, n, vocab), lambda i: (i, 0, 0))],
        out_specs=pl.BlockSpec((_BATCH_PER_BLOCK, n), lambda i: (i, 0)),
        out_shape=jax.ShapeDtypeStruct((b, n), jnp.int32),
    )(x)


def _make_gather(rows, emb, n_workers, n_chunks):
    bpw = rows // n_workers

    def _gather_body(tok_hbm, table_hbm, out_hbm, idx_v, rows_v, sem):
        wid = lax.axis_index("s") * 2 + lax.axis_index("c")
        pltpu.sync_copy(tok_hbm.at[pl.ds(wid * bpw, bpw)], idx_v)
        copies = [
            pltpu.async_copy(
                table_hbm.at[idx_v.at[pl.ds(j * _CHUNK, _CHUNK)]],
                rows_v.at[pl.ds(j * _CHUNK, _CHUNK)],
                sem,
            )
            for j in range(n_chunks)
        ]
        for cp in copies:
            cp.wait()
        pltpu.sync_copy(rows_v, out_hbm.at[pl.ds(wid * bpw, bpw)])

    mesh = plsc.VectorSubcoreMesh(core_axis_name="c", subcore_axis_name="s")
    return pl.kernel(
        _gather_body,
        mesh=mesh,
        compiler_params=pltpu.CompilerParams(use_tc_tiling_on_sc=False),
        out_type=jax.ShapeDtypeStruct((rows, emb), jnp.float32),
        scratch_types=[
            pltpu.VMEM((bpw,), jnp.int32),
            pltpu.VMEM((bpw, emb), jnp.float32),
            pltpu.SemaphoreType.DMA,
        ],
    )


def kernel(x, dictionary):
    b, n, vocab = x.shape
    emb = dictionary.shape[1]
    rows = b * n
    n_workers = 32
    n_chunks = rows // (n_workers * _CHUNK)

    tokens = _compute_tokens(x).reshape(rows)
    out = _make_gather(rows, emb, n_workers, n_chunks)(tokens, dictionary)
    return out.reshape(b, n, emb)


# speedup vs baseline: 1.3335x; 1.0017x over previous
import functools

import jax
import jax.numpy as jnp
from jax import lax
from jax.experimental import pallas as pl
from jax.experimental.pallas import tpu as pltpu
from jax.experimental.pallas import tpu_sc as plsc

_BATCH_PER_BLOCK = 32
_CHUNK = 80


def _argmax_half(xb):
    vocab = xb.shape[-1]
    m = jnp.max(xb, axis=-1, keepdims=True)
    col = jax.lax.broadcasted_iota(jnp.int32, xb.shape, 2)
    return jnp.min(jnp.where(xb == m, col, vocab), axis=-1)


def _argmax_body(xa_ref, xb_ref, ta_ref, tb_ref):
    ta_ref[...] = _argmax_half(xa_ref[...])
    tb_ref[...] = _argmax_half(xb_ref[...])


def _compute_tokens(x):
    b, n, vocab = x.shape
    grid = b // (2 * _BATCH_PER_BLOCK)
    half = b // (2 * _BATCH_PER_BLOCK)
    blk = (_BATCH_PER_BLOCK, n, vocab)
    ta, tb = pl.pallas_call(
        _argmax_body,
        grid=(grid,),
        in_specs=[
            pl.BlockSpec(blk, lambda i: (i, 0, 0)),
            pl.BlockSpec(blk, lambda i, h=half: (i + h, 0, 0)),
        ],
        out_specs=[
            pl.BlockSpec((_BATCH_PER_BLOCK, n), lambda i: (i, 0)),
            pl.BlockSpec((_BATCH_PER_BLOCK, n), lambda i: (i, 0)),
        ],
        out_shape=[
            jax.ShapeDtypeStruct((b // 2, n), jnp.int32),
            jax.ShapeDtypeStruct((b // 2, n), jnp.int32),
        ],
    )(x, x)
    return ta, tb


def _make_gather(rows, emb, n_workers, n_chunks):
    bpw = rows // n_workers

    def _gather_body(tok_hbm, table_hbm, out_hbm, idx_v, rows_v, sem):
        wid = lax.axis_index("s") * 2 + lax.axis_index("c")
        pltpu.sync_copy(tok_hbm.at[pl.ds(wid * bpw, bpw)], idx_v)
        copies = [
            pltpu.async_copy(
                table_hbm.at[idx_v.at[pl.ds(j * _CHUNK, _CHUNK)]],
                rows_v.at[pl.ds(j * _CHUNK, _CHUNK)],
                sem,
            )
            for j in range(n_chunks)
        ]
        for cp in copies:
            cp.wait()
        pltpu.sync_copy(rows_v, out_hbm.at[pl.ds(wid * bpw, bpw)])

    mesh = plsc.VectorSubcoreMesh(core_axis_name="c", subcore_axis_name="s")
    return pl.kernel(
        _gather_body,
        mesh=mesh,
        compiler_params=pltpu.CompilerParams(use_tc_tiling_on_sc=False),
        out_type=jax.ShapeDtypeStruct((rows, emb), jnp.float32),
        scratch_types=[
            pltpu.VMEM((bpw,), jnp.int32),
            pltpu.VMEM((bpw, emb), jnp.float32),
            pltpu.SemaphoreType.DMA,
        ],
    )


def kernel(x, dictionary):
    b, n, vocab = x.shape
    emb = dictionary.shape[1]
    rows = b * n
    n_workers = 32
    n_chunks = rows // (n_workers * _CHUNK)

    ta, tb = _compute_tokens(x)
    tokens = jnp.concatenate([ta.reshape(rows // 2), tb.reshape(rows // 2)])
    out = _make_gather(rows, emb, n_workers, n_chunks)(tokens, dictionary)
    return out.reshape(b, n, emb)
